# initial kernel scaffold (unmeasured)
import jax
import jax.numpy as jnp
from jax import lax
from jax.experimental import pallas as pl
from jax.experimental.pallas import tpu as pltpu


def kernel(
    x,
):
    def body(*refs):
        pass

    out_shape = jax.ShapeDtypeStruct(..., jnp.float32)
    return pl.pallas_call(body, out_shape=out_shape)(...)


# baseline (device time: 334501 ns/iter reference)
import jax
import jax.numpy as jnp
from jax import lax
from jax.experimental import pallas as pl
from jax.experimental.pallas import tpu as pltpu

N_DEV = 4


def kernel(x):
    m_per, n = x.shape
    half = m_per // 2

    def body(x_ref, out_ref, local_sem, send_sems, recv_sems):
        my_pos = lax.axis_index("i")
        left = (my_pos - 1) % N_DEV
        right = (my_pos + 1) % N_DEV

        barrier_sem = pltpu.get_barrier_semaphore()
        for nbr in (left, right):
            pl.semaphore_signal(
                barrier_sem, inc=1,
                device_id=(nbr,), device_id_type=pl.DeviceIdType.MESH,
            )
        pl.semaphore_wait(barrier_sem, 2)

        copy = pltpu.make_async_copy(
            x_ref, out_ref.at[pl.ds(my_pos * m_per, m_per), :], local_sem
        )
        copy.start()
        copy.wait()

        for h in range(N_DEV - 1):
            cw_origin = (my_pos - h) % N_DEV
            ccw_origin = (my_pos + h) % N_DEV
            cw = pltpu.make_async_remote_copy(
                src_ref=out_ref.at[pl.ds(cw_origin * m_per, half), :],
                dst_ref=out_ref.at[pl.ds(cw_origin * m_per, half), :],
                send_sem=send_sems.at[0, h],
                recv_sem=recv_sems.at[0, h],
                device_id=(right,),
                device_id_type=pl.DeviceIdType.MESH,
            )
            ccw = pltpu.make_async_remote_copy(
                src_ref=out_ref.at[pl.ds(ccw_origin * m_per + half, half), :],
                dst_ref=out_ref.at[pl.ds(ccw_origin * m_per + half, half), :],
                send_sem=send_sems.at[1, h],
                recv_sem=recv_sems.at[1, h],
                device_id=(left,),
                device_id_type=pl.DeviceIdType.MESH,
            )
            cw.start()
            ccw.start()
            cw.wait()
            ccw.wait()

    return pl.pallas_call(
        body,
        out_shape=jax.ShapeDtypeStruct((N_DEV * m_per, n), x.dtype),
        in_specs=[pl.BlockSpec(memory_space=pl.ANY)],
        out_specs=pl.BlockSpec(memory_space=pl.ANY),
        scratch_shapes=[
            pltpu.SemaphoreType.DMA,
            pltpu.SemaphoreType.DMA((2, N_DEV - 1)),
            pltpu.SemaphoreType.DMA((2, N_DEV - 1)),
        ],
        compiler_params=pltpu.CompilerParams(collective_id=0),
    )(x)


# device time: 328508 ns/iter; 1.0182x vs baseline; 1.0182x over previous
import jax
import jax.numpy as jnp
from jax import lax
from jax.experimental import pallas as pl
from jax.experimental.pallas import tpu as pltpu

N_DEV = 4


def kernel(x):
    m_per, n = x.shape
    half = m_per // 2

    def body(x_ref, out_ref, local_sem, send_sems, recv_sems):
        my_pos = lax.axis_index("i")
        left = (my_pos - 1) % N_DEV
        right = (my_pos + 1) % N_DEV

        barrier_sem = pltpu.get_barrier_semaphore()
        for nbr in (left, right):
            pl.semaphore_signal(
                barrier_sem, inc=1,
                device_id=(nbr,), device_id_type=pl.DeviceIdType.MESH,
            )
        pl.semaphore_wait(barrier_sem, 2)

        copy = pltpu.make_async_copy(
            x_ref, out_ref.at[pl.ds(my_pos * m_per, m_per), :], local_sem
        )
        copy.start()

        for h in range(N_DEV - 1):
            cw_origin = (my_pos - h) % N_DEV
            ccw_origin = (my_pos + h) % N_DEV
            cw_src = (
                x_ref.at[pl.ds(0, half), :]
                if h == 0
                else out_ref.at[pl.ds(cw_origin * m_per, half), :]
            )
            ccw_src = (
                x_ref.at[pl.ds(half, half), :]
                if h == 0
                else out_ref.at[pl.ds(ccw_origin * m_per + half, half), :]
            )
            cw = pltpu.make_async_remote_copy(
                src_ref=cw_src,
                dst_ref=out_ref.at[pl.ds(cw_origin * m_per, half), :],
                send_sem=send_sems.at[0, h],
                recv_sem=recv_sems.at[0, h],
                device_id=(right,),
                device_id_type=pl.DeviceIdType.MESH,
            )
            ccw = pltpu.make_async_remote_copy(
                src_ref=ccw_src,
                dst_ref=out_ref.at[pl.ds(ccw_origin * m_per + half, half), :],
                send_sem=send_sems.at[1, h],
                recv_sem=recv_sems.at[1, h],
                device_id=(left,),
                device_id_type=pl.DeviceIdType.MESH,
            )
            cw.start()
            ccw.start()
            cw.wait()
            ccw.wait()

        copy.wait()

    return pl.pallas_call(
        body,
        out_shape=jax.ShapeDtypeStruct((N_DEV * m_per, n), x.dtype),
        in_specs=[pl.BlockSpec(memory_space=pl.ANY)],
        out_specs=pl.BlockSpec(memory_space=pl.ANY),
        scratch_shapes=[
            pltpu.SemaphoreType.DMA,
            pltpu.SemaphoreType.DMA((2, N_DEV - 1)),
            pltpu.SemaphoreType.DMA((2, N_DEV - 1)),
        ],
        compiler_params=pltpu.CompilerParams(collective_id=0),
    )(x)


# device time: 324731 ns/iter; 1.0301x vs baseline; 1.0116x over previous
import jax
import jax.numpy as jnp
from jax import lax
from jax.experimental import pallas as pl
from jax.experimental.pallas import tpu as pltpu

N_DEV = 4
SUBS = 4


def kernel(x):
    m_per, n = x.shape
    half = m_per // 2
    sub = half // SUBS

    def body(x_ref, out_ref, local_sem, send_sems, recv_sems):
        my_pos = lax.axis_index("i")
        left = (my_pos - 1) % N_DEV
        right = (my_pos + 1) % N_DEV

        def cw_rows(origin, k):
            return pl.ds(origin * m_per + k * sub, sub)

        def ccw_rows(origin, k):
            return pl.ds(origin * m_per + half + k * sub, sub)

        def cw_rdma(h, k):
            origin = (my_pos - h) % N_DEV
            src = (
                x_ref.at[pl.ds(k * sub, sub), :]
                if h == 0
                else out_ref.at[cw_rows(origin, k), :]
            )
            return pltpu.make_async_remote_copy(
                src_ref=src,
                dst_ref=out_ref.at[cw_rows(origin, k), :],
                send_sem=send_sems.at[0, h, k],
                recv_sem=recv_sems.at[0, h, k],
                device_id=(right,),
                device_id_type=pl.DeviceIdType.MESH,
            )

        def ccw_rdma(h, k):
            origin = (my_pos + h) % N_DEV
            src = (
                x_ref.at[pl.ds(half + k * sub, sub), :]
                if h == 0
                else out_ref.at[ccw_rows(origin, k), :]
            )
            return pltpu.make_async_remote_copy(
                src_ref=src,
                dst_ref=out_ref.at[ccw_rows(origin, k), :],
                send_sem=send_sems.at[1, h, k],
                recv_sem=recv_sems.at[1, h, k],
                device_id=(left,),
                device_id_type=pl.DeviceIdType.MESH,
            )

        barrier_sem = pltpu.get_barrier_semaphore()
        for nbr in (left, right):
            pl.semaphore_signal(
                barrier_sem, inc=1,
                device_id=(nbr,), device_id_type=pl.DeviceIdType.MESH,
            )
        pl.semaphore_wait(barrier_sem, 2)

        copy = pltpu.make_async_copy(
            x_ref, out_ref.at[pl.ds(my_pos * m_per, m_per), :], local_sem
        )
        copy.start()

        for h in range(N_DEV - 1):
            for k in range(SUBS):
                if h > 0:
                    cw_rdma(h - 1, k).wait_recv()
                    ccw_rdma(h - 1, k).wait_recv()
                cw_rdma(h, k).start()
                ccw_rdma(h, k).start()

        for k in range(SUBS):
            cw_rdma(N_DEV - 2, k).wait_recv()
            ccw_rdma(N_DEV - 2, k).wait_recv()
        for h in range(N_DEV - 1):
            for k in range(SUBS):
                cw_rdma(h, k).wait_send()
                ccw_rdma(h, k).wait_send()

        copy.wait()

    return pl.pallas_call(
        body,
        out_shape=jax.ShapeDtypeStruct((N_DEV * m_per, n), x.dtype),
        in_specs=[pl.BlockSpec(memory_space=pl.ANY)],
        out_specs=pl.BlockSpec(memory_space=pl.ANY),
        scratch_shapes=[
            pltpu.SemaphoreType.DMA,
            pltpu.SemaphoreType.DMA((2, N_DEV - 1, SUBS)),
            pltpu.SemaphoreType.DMA((2, N_DEV - 1, SUBS)),
        ],
        compiler_params=pltpu.CompilerParams(collective_id=0),
    )(x)


# device time: 323722 ns/iter; 1.0333x vs baseline; 1.0031x over previous
import jax
import jax.numpy as jnp
from jax import lax
from jax.experimental import pallas as pl
from jax.experimental.pallas import tpu as pltpu

N_DEV = 4
SUBS = 4


def kernel(x):
    m_per, n = x.shape
    half = m_per // 2
    sub = half // SUBS

    def body(x_ref, out_ref, local_sem, send_sems, recv_sems):
        my_pos = lax.axis_index("i")
        left = (my_pos - 1) % N_DEV
        right = (my_pos + 1) % N_DEV

        def cw_rows(origin, k):
            return pl.ds(origin * m_per + k * sub, sub)

        def ccw_rows(origin, k):
            return pl.ds(origin * m_per + half + k * sub, sub)

        def cw_rdma(h, k):
            origin = (my_pos - h) % N_DEV
            src = (
                x_ref.at[pl.ds(k * sub, sub), :]
                if h == 0
                else out_ref.at[cw_rows(origin, k), :]
            )
            return pltpu.make_async_remote_copy(
                src_ref=src,
                dst_ref=out_ref.at[cw_rows(origin, k), :],
                send_sem=send_sems.at[0, h, k],
                recv_sem=recv_sems.at[0, h, k],
                device_id=(right,),
                device_id_type=pl.DeviceIdType.MESH,
            )

        def ccw_rdma(h, k):
            origin = (my_pos + h) % N_DEV
            src = (
                x_ref.at[pl.ds(half + k * sub, sub), :]
                if h == 0
                else out_ref.at[ccw_rows(origin, k), :]
            )
            return pltpu.make_async_remote_copy(
                src_ref=src,
                dst_ref=out_ref.at[ccw_rows(origin, k), :],
                send_sem=send_sems.at[1, h, k],
                recv_sem=recv_sems.at[1, h, k],
                device_id=(left,),
                device_id_type=pl.DeviceIdType.MESH,
            )

        barrier_sem = pltpu.get_barrier_semaphore()
        for nbr in (left, right):
            pl.semaphore_signal(
                barrier_sem, inc=1,
                device_id=(nbr,), device_id_type=pl.DeviceIdType.MESH,
            )
        pl.semaphore_wait(barrier_sem, 2)

        copy = pltpu.make_async_copy(
            x_ref, out_ref.at[pl.ds(my_pos * m_per, m_per), :], local_sem
        )
        copy.start()

        for h in range(N_DEV - 1):
            for k in range(SUBS):
                if h > 0:
                    cw_rdma(h - 1, k).wait_recv()
                    ccw_rdma(h - 1, k).wait_recv()
                cw_rdma(h, k).start()
                ccw_rdma(h, k).start()

        for k in range(SUBS):
            cw_rdma(N_DEV - 2, k).wait_recv()
            ccw_rdma(N_DEV - 2, k).wait_recv()
        for h in range(N_DEV - 1):
            for k in range(SUBS):
                cw_rdma(h, k).wait_send()
                ccw_rdma(h, k).wait_send()

        copy.wait()

    return pl.pallas_call(
        body,
        out_shape=jax.ShapeDtypeStruct((N_DEV * m_per, n), x.dtype),
        in_specs=[pl.BlockSpec(memory_space=pltpu.MemorySpace.VMEM)],
        out_specs=pl.BlockSpec(memory_space=pl.ANY),
        scratch_shapes=[
            pltpu.SemaphoreType.DMA,
            pltpu.SemaphoreType.DMA((2, N_DEV - 1, SUBS)),
            pltpu.SemaphoreType.DMA((2, N_DEV - 1, SUBS)),
        ],
        compiler_params=pltpu.CompilerParams(collective_id=0),
    )(x)
